# Initial kernel scaffold; baseline (speedup 1.0000x reference)
import jax
import jax.numpy as jnp
from jax.experimental import pallas as pl


def kernel(obs, prev_full_action_wk, W1, b1, W2, b2, W3, b3, walking_action_out_indices, walking_offsets_indices, walking_offsets, walking_defaults, keep_mask):
    raise NotImplementedError("write your pallas kernel here")


# speedup vs baseline: 1.1642x; 1.1642x over previous
import jax
import jax.numpy as jnp
from jax.experimental import pallas as pl
from jax.experimental.pallas import tpu as pltpu

_ACTION_S_IDX = 68
_ACTION_E_IDX = 88
_FULL_ACTION_DIM = 28
_PAD = 128
_BM = 1024


def _fused_body(obs_ref, prev_ref, w1_ref, b1_ref, w2_ref, b2_ref, w3_ref,
                t_ref, tb_ref, out_ref):
    obs = obs_ref[...]
    col = jax.lax.broadcasted_iota(jnp.int32, obs.shape, 1)
    in_seg = (col >= _ACTION_S_IDX) & (col < _ACTION_E_IDX)
    x = jnp.where(in_seg, prev_ref[...], obs)
    h = jnp.tanh(jnp.dot(x, w1_ref[...], preferred_element_type=jnp.float32)
                 + b1_ref[...])
    h = jnp.tanh(jnp.dot(h, w2_ref[...], preferred_element_type=jnp.float32)
                 + b2_ref[...])
    a = jnp.dot(h, w3_ref[...], preferred_element_type=jnp.float32)
    res = jnp.dot(a, t_ref[...], preferred_element_type=jnp.float32) + tb_ref[...]
    out_ref[...] = res[:, :out_ref.shape[1]]


def kernel(obs, prev_full_action_wk, W1, b1, W2, b2, W3, b3,
           walking_action_out_indices, walking_offsets_indices,
           walking_offsets, walking_defaults, keep_mask):
    B, D = obs.shape
    H = W1.shape[1]
    nact = W3.shape[1]
    nkeep = keep_mask.shape[0]
    outw = nkeep + 4
    f32 = jnp.float32

    prev_row = jax.lax.dynamic_update_slice(
        jnp.zeros((1, D), f32), prev_full_action_wk.astype(f32),
        (0, _ACTION_S_IDX))

    M = jnp.zeros((nact, _FULL_ACTION_DIM), f32)
    M = M.at[jnp.arange(nact), walking_action_out_indices].set(0.1)
    c = jnp.zeros((_FULL_ACTION_DIM,), f32)
    c = c.at[walking_offsets_indices].set(walking_defaults)
    c = c.at[walking_action_out_indices].set(walking_offsets)
    T = jnp.zeros((_PAD, _PAD), f32).at[:nact, :nkeep].set(M[:, keep_mask])
    W3p = jnp.zeros((H, _PAD), f32).at[:, :nact].set(W3)
    b3p = jnp.zeros((_PAD,), f32).at[:nact].set(b3)
    tb = (b3p @ T + jnp.zeros((_PAD,), f32).at[:nkeep].set(c[keep_mask]))
    tb = tb.reshape(1, _PAD)

    bm = min(_BM, B)
    out = pl.pallas_call(
        _fused_body,
        grid=(pl.cdiv(B, bm),),
        in_specs=[
            pl.BlockSpec((bm, D), lambda i: (i, 0)),
            pl.BlockSpec((1, D), lambda i: (0, 0)),
            pl.BlockSpec((D, H), lambda i: (0, 0)),
            pl.BlockSpec((1, H), lambda i: (0, 0)),
            pl.BlockSpec((H, H), lambda i: (0, 0)),
            pl.BlockSpec((1, H), lambda i: (0, 0)),
            pl.BlockSpec((H, _PAD), lambda i: (0, 0)),
            pl.BlockSpec((_PAD, _PAD), lambda i: (0, 0)),
            pl.BlockSpec((1, _PAD), lambda i: (0, 0)),
        ],
        out_specs=pl.BlockSpec((bm, outw), lambda i: (i, 0)),
        out_shape=jax.ShapeDtypeStruct((B, outw), f32),
        compiler_params=pltpu.CompilerParams(
            dimension_semantics=("arbitrary",)),
    )(obs, prev_row, W1, b1.reshape(1, H), W2, b2.reshape(1, H), W3p, T, tb)
    return out
